# baseline (speedup 1.0000x reference)
import functools

import jax
import jax.numpy as jnp
from jax import lax
from jax.experimental import pallas as pl
from jax.experimental.pallas import tpu as pltpu
from jax.experimental.pallas import tpu_sc as plsc


def _pos_embed_sc(row_embed, col_embed, *, b, h, w, d):
    info = plsc.get_sparse_core_info()
    nc, ns, lanes = info.num_cores, info.num_subcores, info.num_lanes
    nw = nc * ns
    dchunks = d // lanes

    mesh = plsc.VectorSubcoreMesh(core_axis_name="c", subcore_axis_name="s")

    @functools.partial(
        pl.kernel,
        out_type=jax.ShapeDtypeStruct((b, h, w, d), jnp.float32),
        mesh=mesh,
        scratch_types=[
            pltpu.VMEM((8, d), jnp.float32),
            pltpu.VMEM((w, d), jnp.float32),
            pltpu.VMEM((w, d), jnp.float32),
            pltpu.VMEM_SHARED((ns, w, d), jnp.float32),
            pltpu.SemaphoreType.DMA,
            pltpu.SemaphoreType.DMA,
        ],
        compiler_params=pltpu.CompilerParams(
            use_tc_tiling_on_sc=True, needs_layout_passes=False
        ),
    )
    def body(
        row_hbm, col_hbm, out_hbm, row_v, col_v, blk_v, shared, stage_sem, sem
    ):
        c = lax.axis_index("c")
        s = lax.axis_index("s")
        i = c * ns + s

        g0 = (i // 8) * 8
        hw = w // 2
        st_r = pltpu.async_copy(
            row_hbm.at[pl.ds(g0, 8), :], row_v, stage_sem
        )
        st_c = [
            pltpu.async_copy(
                col_hbm.at[pl.ds(half * hw, hw), :],
                col_v.at[pl.ds(half * hw, hw), :],
                stage_sem,
            )
            for half in range(2)
        ]
        st_r.wait()

        ii = i - g0
        r_chunks = [row_v[ii, pl.ds(k * lanes, lanes)] for k in range(dchunks)]

        def jrow(j, _):
            for k in range(dchunks):
                sl = pl.ds(k * lanes, lanes)
                blk_v[j, sl] = col_v[j, sl] + r_chunks[k]
            return 0

        pubs = []
        for half in range(2):
            st_c[half].wait()
            lax.fori_loop(half * hw, (half + 1) * hw, jrow, 0)
            pubs.append(
                pltpu.async_copy(
                    blk_v.at[pl.ds(half * hw, hw), :],
                    shared.at[s, pl.ds(half * hw, hw)],
                    sem,
                )
            )
        for p in pubs:
            p.wait()
        plsc.subcore_barrier()

        @pl.when(s < b)
        def _():
            pltpu.sync_copy(shared, out_hbm.at[s, pl.ds(c * ns, ns)])

    return body(row_embed, col_embed)


def kernel(x, row_embed, col_embed):
    b = x.shape[0]
    h, w = x.shape[-2], x.shape[-1]
    d = row_embed.shape[1]
    out_bhwd = _pos_embed_sc(row_embed, col_embed, b=b, h=h, w=w, d=d)
    return jnp.transpose(out_bhwd, (0, 3, 1, 2))


# speedup vs baseline: 1.0587x; 1.0587x over previous
import functools

import jax
import jax.numpy as jnp
from jax import lax
from jax.experimental import pallas as pl
from jax.experimental.pallas import tpu as pltpu
from jax.experimental.pallas import tpu_sc as plsc


def _pos_embed_sc(row_embed, col_embed, *, b, h, w, d):
    info = plsc.get_sparse_core_info()
    nc, ns, lanes = info.num_cores, info.num_subcores, info.num_lanes
    nw = nc * ns
    dchunks = d // lanes

    mesh = plsc.VectorSubcoreMesh(core_axis_name="c", subcore_axis_name="s")

    @functools.partial(
        pl.kernel,
        out_type=jax.ShapeDtypeStruct((b, h, w, d), jnp.float32),
        mesh=mesh,
        scratch_types=[
            pltpu.VMEM((8, d), jnp.float32),
            pltpu.VMEM((w, d), jnp.float32),
            pltpu.VMEM((w, d), jnp.float32),
            pltpu.SemaphoreType.DMA,
            pltpu.SemaphoreType.DMA,
        ],
        compiler_params=pltpu.CompilerParams(
            use_tc_tiling_on_sc=True, needs_layout_passes=False
        ),
    )
    def body(row_hbm, col_hbm, out_hbm, row_v, col_v, blk_v, stage_sem, sem):
        i = lax.axis_index("s") * nc + lax.axis_index("c")

        g0 = (i // 8) * 8
        hw = w // 2
        st_r = pltpu.async_copy(
            row_hbm.at[pl.ds(g0, 8), :], row_v, stage_sem
        )
        st_c = [
            pltpu.async_copy(
                col_hbm.at[pl.ds(half * hw, hw), :],
                col_v.at[pl.ds(half * hw, hw), :],
                stage_sem,
            )
            for half in range(2)
        ]
        st_r.wait()

        ii = i - g0
        r_chunks = [row_v[ii, pl.ds(k * lanes, lanes)] for k in range(dchunks)]

        def jrow(j, _):
            for k in range(dchunks):
                sl = pl.ds(k * lanes, lanes)
                blk_v[j, sl] = col_v[j, sl] + r_chunks[k]
            return 0

        copies = []
        for half in range(2):
            st_c[half].wait()
            lax.fori_loop(half * hw, (half + 1) * hw, jrow, 0)
            src = blk_v.at[pl.ds(half * hw, hw), :]
            copies += [
                pltpu.async_copy(
                    src, out_hbm.at[bi, i, pl.ds(half * hw, hw)], sem
                )
                for bi in range(b)
            ]
        for c in copies:
            c.wait()

    return body(row_embed, col_embed)


def kernel(x, row_embed, col_embed):
    b = x.shape[0]
    h, w = x.shape[-2], x.shape[-1]
    d = row_embed.shape[1]
    out_bhwd = _pos_embed_sc(row_embed, col_embed, b=b, h=h, w=w, d=d)
    return jnp.transpose(out_bhwd, (0, 3, 1, 2))
